# baseline (speedup 1.0000x reference)
import functools

import jax
import jax.numpy as jnp
from jax import lax
from jax.experimental import pallas as pl
from jax.experimental.pallas import tpu as pltpu
from jax.experimental.pallas import tpu_sc as plsc

N = 10000
E = 320000
D = 128
DE = 16

NC = 2
NS = 16
NW = NC * NS
EW = E // NW
B = 40
NB = EW // B
SB = 50
NSB = NB // SB
ST = 624
TAIL = N - NS * ST


def _ea_body(ea_ref, we_ref, be_ref, out_ref):
    out_ref[...] = (
        jnp.dot(ea_ref[...], we_ref[...], preferred_element_type=jnp.float32)
        + be_ref[...]
    )


def _compute_ea(edge_attr, W_e, b_e):
    BE = 16000
    return pl.pallas_call(
        _ea_body,
        grid=(E // BE,),
        in_specs=[
            pl.BlockSpec((BE, DE), lambda i: (i, 0)),
            pl.BlockSpec((DE, D), lambda i: (0, 0)),
            pl.BlockSpec((1, D), lambda i: (0, 0)),
        ],
        out_specs=pl.BlockSpec((BE, D), lambda i: (i, 0)),
        out_shape=jax.ShapeDtypeStruct((E, D), jnp.float32),
    )(edge_attr, W_e, b_e.reshape(1, D))


def _sc_body(x_hbm, ea_hbm, src_hbm, dst_hbm,
             aggp_hbm, cntp_hbm,
             src_v, dst_v, xj0, xj1, ea0, ea1, ones_v, zb16,
             agg_sh, cnt_sh,
             in_s0, in_s1, out_s0, out_s1):
    c = lax.axis_index("c")
    s = lax.axis_index("s")
    wid = c * NS + s

    xj = (xj0, xj1)
    eab = (ea0, ea1)
    in_s = (in_s0, in_s1)
    out_s = (out_s0, out_s1)

    for i in range(B):
        for j in range(D // 16):
            xj0[i, pl.ds(j * 16, 16)] = jnp.zeros((16,), jnp.float32)
        zb16[i, :] = jnp.zeros((16,), jnp.float32)
        ones_v[i, :] = jnp.ones((16,), jnp.float32)

    NZ = ST // B
    REM = ST - NZ * B
    for k in range(NZ):
        pltpu.async_copy(xj0, agg_sh.at[pl.ds(s * ST + k * B, B)], in_s0)
        pltpu.async_copy(zb16, cnt_sh.at[pl.ds(s * ST + k * B, B)], in_s1)
    if REM:
        pltpu.async_copy(xj0.at[pl.ds(0, REM)],
                         agg_sh.at[pl.ds(s * ST + NZ * B, REM)], in_s0)
        pltpu.async_copy(zb16.at[pl.ds(0, REM)],
                         cnt_sh.at[pl.ds(s * ST + NZ * B, REM)], in_s1)

    @pl.when(s == NS - 1)
    def _zero_tail():
        pltpu.async_copy(xj0.at[pl.ds(0, TAIL)],
                         agg_sh.at[pl.ds(NS * ST, TAIL)], out_s0)
        pltpu.async_copy(zb16.at[pl.ds(0, TAIL)],
                         cnt_sh.at[pl.ds(NS * ST, TAIL)], out_s1)

    for k in range(NZ):
        pltpu.make_async_copy(x_hbm.at[pl.ds(0, B)], xj0, in_s0).wait()
        pltpu.make_async_copy(cntp_hbm.at[0, pl.ds(0, B)], zb16,
                              in_s1).wait()
    if REM:
        pltpu.make_async_copy(x_hbm.at[pl.ds(0, REM)],
                              xj0.at[pl.ds(0, REM)], in_s0).wait()
        pltpu.make_async_copy(cntp_hbm.at[0, pl.ds(0, REM)],
                              zb16.at[pl.ds(0, REM)], in_s1).wait()

    @pl.when(s == NS - 1)
    def _zero_tail_wait():
        pltpu.make_async_copy(x_hbm.at[pl.ds(0, TAIL)],
                              xj0.at[pl.ds(0, TAIL)], out_s0).wait()
        pltpu.make_async_copy(cntp_hbm.at[0, pl.ds(0, TAIL)],
                              zb16.at[pl.ds(0, TAIL)], out_s1).wait()

    plsc.subcore_barrier()

    def start_in(g, p):
        pltpu.async_copy(x_hbm.at[src_v.at[g % SB]], xj[p], in_s[p])
        off = pl.multiple_of(wid * EW + g * B, 8)
        pltpu.async_copy(ea_hbm.at[pl.ds(off, B)], eab[p], in_s[p])

    def wait_in(p):
        pltpu.make_async_copy(x_hbm.at[pl.ds(0, B)], xj[p], in_s[p]).wait()
        pltpu.make_async_copy(x_hbm.at[pl.ds(0, B)], eab[p], in_s[p]).wait()

    def start_out(g, p):
        pltpu.async_copy(xj[p], agg_sh.at[dst_v.at[g % SB]], out_s[p],
                         add=True)
        pltpu.async_copy(ones_v, cnt_sh.at[dst_v.at[g % SB]], out_s[p],
                         add=True)

    def wait_out(p):
        pltpu.make_async_copy(x_hbm.at[pl.ds(0, B)], xj[p], out_s[p]).wait()
        pltpu.make_async_copy(cntp_hbm.at[0, pl.ds(0, B)], ones_v,
                              out_s[p]).wait()

    pltpu.sync_copy(src_hbm.at[wid, pl.ds(0, SB)], src_v)
    pltpu.sync_copy(dst_hbm.at[wid, pl.ds(0, SB)], dst_v)
    start_in(0, 0)

    def phase(g, p):
        q = 1 - p
        wait_in(p)

        @pl.when(g + 1 < NB)
        def _next_in():
            @pl.when((g + 1) % SB == 0)
            def _stage_src():
                pltpu.sync_copy(src_hbm.at[wid, pl.ds(g + 1, SB)], src_v)

            @pl.when(g >= 1)
            def _drain_prev():
                wait_out(q)

            start_in(g + 1, q)

        for i in range(B):
            for j in range(D // 16):
                sl = pl.ds(j * 16, 16)
                xj[p][i, sl] = xj[p][i, sl] * eab[p][i, sl]

        @pl.when(jnp.logical_and(g % SB == 0, g >= 1))
        def _stage_dst():
            pltpu.sync_copy(dst_hbm.at[wid, pl.ds(g, SB)], dst_v)

        start_out(g, p)

    def pair(g2, carry):
        g = g2 * 2
        phase(g, 0)
        phase(g + 1, 1)
        return carry

    lax.fori_loop(0, NB // 2, pair, 0)
    wait_out(0)
    wait_out(1)

    plsc.subcore_barrier()

    pltpu.async_copy(agg_sh.at[pl.ds(s * ST, ST)],
                     aggp_hbm.at[c, pl.ds(s * ST, ST)], in_s0)
    pltpu.async_copy(cnt_sh.at[pl.ds(s * ST, ST)],
                     cntp_hbm.at[c, pl.ds(s * ST, ST)], in_s1)

    @pl.when(s == NS - 1)
    def _dump_tail():
        pltpu.async_copy(agg_sh.at[pl.ds(NS * ST, TAIL)],
                         aggp_hbm.at[c, pl.ds(NS * ST, TAIL)], out_s0)
        pltpu.async_copy(cnt_sh.at[pl.ds(NS * ST, TAIL)],
                         cntp_hbm.at[c, pl.ds(NS * ST, TAIL)], out_s1)

    pltpu.make_async_copy(aggp_hbm.at[c, pl.ds(0, ST)],
                          agg_sh.at[pl.ds(s * ST, ST)], in_s0).wait()
    pltpu.make_async_copy(cntp_hbm.at[c, pl.ds(0, ST)],
                          cnt_sh.at[pl.ds(s * ST, ST)], in_s1).wait()

    @pl.when(s == NS - 1)
    def _dump_tail_wait():
        pltpu.make_async_copy(aggp_hbm.at[c, pl.ds(0, TAIL)],
                              agg_sh.at[pl.ds(NS * ST, TAIL)], out_s0).wait()
        pltpu.make_async_copy(cntp_hbm.at[c, pl.ds(0, TAIL)],
                              cnt_sh.at[pl.ds(NS * ST, TAIL)], out_s1).wait()


def _segment_mean_sums(x, ea, src_r, dst_r):
    mesh = plsc.VectorSubcoreMesh(core_axis_name="c", subcore_axis_name="s")
    f = pl.kernel(
        _sc_body,
        out_type=[
            jax.ShapeDtypeStruct((NC, N, D), jnp.float32),
            jax.ShapeDtypeStruct((NC, N, 16), jnp.float32),
        ],
        mesh=mesh,
        scratch_types=[
            pltpu.VMEM((SB, B), jnp.int32),
            pltpu.VMEM((SB, B), jnp.int32),
            pltpu.VMEM((B, D), jnp.float32),
            pltpu.VMEM((B, D), jnp.float32),
            pltpu.VMEM((B, D), jnp.float32),
            pltpu.VMEM((B, D), jnp.float32),
            pltpu.VMEM((B, 16), jnp.float32),
            pltpu.VMEM((B, 16), jnp.float32),
            pltpu.VMEM_SHARED((N, D), jnp.float32),
            pltpu.VMEM_SHARED((N, 16), jnp.float32),
            pltpu.SemaphoreType.DMA,
            pltpu.SemaphoreType.DMA,
            pltpu.SemaphoreType.DMA,
            pltpu.SemaphoreType.DMA,
        ],
        compiler_params=pltpu.CompilerParams(use_tc_tiling_on_sc=False),
    )
    return f(x, ea, src_r, dst_r)


def _out_body(aggp_ref, cntp_ref, x_ref, wl_ref, bl_ref, wr_ref, out_ref):
    agg = aggp_ref[0] + aggp_ref[1]
    cnt = (cntp_ref[0] + cntp_ref[1]).sum(axis=1) * (1.0 / 16.0)
    agg = agg / jnp.clip(cnt, 1.0)[:, None]
    out_ref[...] = (
        jnp.dot(agg, wl_ref[...], preferred_element_type=jnp.float32)
        + bl_ref[...]
        + jnp.dot(x_ref[...], wr_ref[...], preferred_element_type=jnp.float32)
    )


def _final(aggp, cntp, x, W_l, b_l, W_r):
    BN = 2000
    return pl.pallas_call(
        _out_body,
        grid=(N // BN,),
        in_specs=[
            pl.BlockSpec((NC, BN, D), lambda i: (0, i, 0)),
            pl.BlockSpec((NC, BN, 16), lambda i: (0, i, 0)),
            pl.BlockSpec((BN, D), lambda i: (i, 0)),
            pl.BlockSpec((D, D), lambda i: (0, 0)),
            pl.BlockSpec((1, D), lambda i: (0, 0)),
            pl.BlockSpec((D, D), lambda i: (0, 0)),
        ],
        out_specs=pl.BlockSpec((BN, D), lambda i: (i, 0)),
        out_shape=jax.ShapeDtypeStruct((N, D), jnp.float32),
    )(aggp, cntp, x, W_l, b_l.reshape(1, D), W_r)


@jax.jit
def kernel(x, edge_attr, edge_index, W_l, b_l, W_r, W_e, b_e):
    ea = _compute_ea(edge_attr, W_e, b_e)
    src_r = edge_index[0].reshape(NW, NB, B)
    dst_r = edge_index[1].reshape(NW, NB, B)
    aggp, cntp = _segment_mean_sums(x, ea, src_r, dst_r)
    out = _final(aggp, cntp, x, W_l, b_l, W_r)
    return (out, ea)


# speedup vs baseline: 1.0031x; 1.0031x over previous
import functools

import jax
import jax.numpy as jnp
from jax import lax
from jax.experimental import pallas as pl
from jax.experimental.pallas import tpu as pltpu
from jax.experimental.pallas import tpu_sc as plsc

N = 10000
E = 320000
D = 128
DE = 16

NC = 2
NS = 16
NW = NC * NS
EW = E // NW
B = 40
NB = EW // B
SB = 50
NSB = NB // SB
ST = 624
TAIL = N - NS * ST


def _ea_body(ea_ref, we_ref, be_ref, out_ref):
    out_ref[...] = (
        jnp.dot(ea_ref[...], we_ref[...], preferred_element_type=jnp.float32)
        + be_ref[...]
    )


def _compute_ea(edge_attr, W_e, b_e):
    BE = 8000
    return pl.pallas_call(
        _ea_body,
        grid=(E // BE,),
        in_specs=[
            pl.BlockSpec((BE, DE), lambda i: (i, 0)),
            pl.BlockSpec((DE, D), lambda i: (0, 0)),
            pl.BlockSpec((1, D), lambda i: (0, 0)),
        ],
        out_specs=pl.BlockSpec((BE, D), lambda i: (i, 0)),
        out_shape=jax.ShapeDtypeStruct((E, D), jnp.float32),
    )(edge_attr, W_e, b_e.reshape(1, D))


def _sc_body(x_hbm, ea_hbm, src_hbm, dst_hbm,
             aggp_hbm, cntp_hbm,
             src_v, dst_v, xj0, xj1, ea0, ea1, ones_v, zb16,
             agg_sh, cnt_sh,
             in_s0, in_s1, out_s0, out_s1):
    c = lax.axis_index("c")
    s = lax.axis_index("s")
    wid = c * NS + s

    xj = (xj0, xj1)
    eab = (ea0, ea1)
    in_s = (in_s0, in_s1)
    out_s = (out_s0, out_s1)

    for i in range(B):
        for j in range(D // 16):
            xj0[i, pl.ds(j * 16, 16)] = jnp.zeros((16,), jnp.float32)
        zb16[i, :] = jnp.zeros((16,), jnp.float32)
        ones_v[i, :] = jnp.ones((16,), jnp.float32)

    NZ = ST // B
    REM = ST - NZ * B
    for k in range(NZ):
        pltpu.sync_copy(xj0, agg_sh.at[pl.ds(s * ST + k * B, B)])
        pltpu.sync_copy(zb16, cnt_sh.at[pl.ds(s * ST + k * B, B)])
    if REM:
        pltpu.sync_copy(xj0.at[pl.ds(0, REM)],
                        agg_sh.at[pl.ds(s * ST + NZ * B, REM)])
        pltpu.sync_copy(zb16.at[pl.ds(0, REM)],
                        cnt_sh.at[pl.ds(s * ST + NZ * B, REM)])

    @pl.when(s == NS - 1)
    def _zero_tail():
        pltpu.sync_copy(xj0.at[pl.ds(0, TAIL)],
                        agg_sh.at[pl.ds(NS * ST, TAIL)])
        pltpu.sync_copy(zb16.at[pl.ds(0, TAIL)],
                        cnt_sh.at[pl.ds(NS * ST, TAIL)])

    plsc.subcore_barrier()

    def start_in(g, p):
        pltpu.async_copy(x_hbm.at[src_v.at[g % SB]], xj[p], in_s[p])
        off = pl.multiple_of(wid * EW + g * B, 8)
        pltpu.async_copy(ea_hbm.at[pl.ds(off, B)], eab[p], in_s[p])

    def wait_in(p):
        pltpu.make_async_copy(x_hbm.at[pl.ds(0, B)], xj[p], in_s[p]).wait()
        pltpu.make_async_copy(x_hbm.at[pl.ds(0, B)], eab[p], in_s[p]).wait()

    def start_out(g, p):
        pltpu.async_copy(xj[p], agg_sh.at[dst_v.at[g % SB]], out_s[p],
                         add=True)
        pltpu.async_copy(ones_v, cnt_sh.at[dst_v.at[g % SB]], out_s[p],
                         add=True)

    def wait_out(p):
        pltpu.make_async_copy(x_hbm.at[pl.ds(0, B)], xj[p], out_s[p]).wait()
        pltpu.make_async_copy(cntp_hbm.at[0, pl.ds(0, B)], ones_v,
                              out_s[p]).wait()

    pltpu.sync_copy(src_hbm.at[wid, pl.ds(0, SB)], src_v)
    pltpu.sync_copy(dst_hbm.at[wid, pl.ds(0, SB)], dst_v)
    start_in(0, 0)

    def phase(g, p):
        q = 1 - p
        wait_in(p)

        @pl.when(g + 1 < NB)
        def _next_in():
            @pl.when((g + 1) % SB == 0)
            def _stage_src():
                pltpu.sync_copy(src_hbm.at[wid, pl.ds(g + 1, SB)], src_v)

            @pl.when(g >= 1)
            def _drain_prev():
                wait_out(q)

            start_in(g + 1, q)

        for i in range(B):
            for j in range(D // 16):
                sl = pl.ds(j * 16, 16)
                xj[p][i, sl] = xj[p][i, sl] * eab[p][i, sl]

        @pl.when(jnp.logical_and(g % SB == 0, g >= 1))
        def _stage_dst():
            pltpu.sync_copy(dst_hbm.at[wid, pl.ds(g, SB)], dst_v)

        start_out(g, p)

    def pair(g2, carry):
        g = g2 * 2
        phase(g, 0)
        phase(g + 1, 1)
        return carry

    lax.fori_loop(0, NB // 2, pair, 0)
    wait_out(0)
    wait_out(1)

    plsc.subcore_barrier()

    pltpu.sync_copy(agg_sh.at[pl.ds(s * ST, ST)],
                    aggp_hbm.at[c, pl.ds(s * ST, ST)])
    pltpu.sync_copy(cnt_sh.at[pl.ds(s * ST, ST)],
                    cntp_hbm.at[c, pl.ds(s * ST, ST)])

    @pl.when(s == NS - 1)
    def _dump_tail():
        pltpu.sync_copy(agg_sh.at[pl.ds(NS * ST, TAIL)],
                        aggp_hbm.at[c, pl.ds(NS * ST, TAIL)])
        pltpu.sync_copy(cnt_sh.at[pl.ds(NS * ST, TAIL)],
                        cntp_hbm.at[c, pl.ds(NS * ST, TAIL)])


def _segment_mean_sums(x, ea, src_r, dst_r):
    mesh = plsc.VectorSubcoreMesh(core_axis_name="c", subcore_axis_name="s")
    f = pl.kernel(
        _sc_body,
        out_type=[
            jax.ShapeDtypeStruct((NC, N, D), jnp.float32),
            jax.ShapeDtypeStruct((NC, N, 16), jnp.float32),
        ],
        mesh=mesh,
        scratch_types=[
            pltpu.VMEM((SB, B), jnp.int32),
            pltpu.VMEM((SB, B), jnp.int32),
            pltpu.VMEM((B, D), jnp.float32),
            pltpu.VMEM((B, D), jnp.float32),
            pltpu.VMEM((B, D), jnp.float32),
            pltpu.VMEM((B, D), jnp.float32),
            pltpu.VMEM((B, 16), jnp.float32),
            pltpu.VMEM((B, 16), jnp.float32),
            pltpu.VMEM_SHARED((N, D), jnp.float32),
            pltpu.VMEM_SHARED((N, 16), jnp.float32),
            pltpu.SemaphoreType.DMA,
            pltpu.SemaphoreType.DMA,
            pltpu.SemaphoreType.DMA,
            pltpu.SemaphoreType.DMA,
        ],
        compiler_params=pltpu.CompilerParams(use_tc_tiling_on_sc=False),
    )
    return f(x, ea, src_r, dst_r)


def _out_body(aggp_ref, cntp_ref, x_ref, wl_ref, bl_ref, wr_ref, out_ref):
    agg = aggp_ref[0] + aggp_ref[1]
    cnt = (cntp_ref[0] + cntp_ref[1]).sum(axis=1) * (1.0 / 16.0)
    agg = agg / jnp.clip(cnt, 1.0)[:, None]
    out_ref[...] = (
        jnp.dot(agg, wl_ref[...], preferred_element_type=jnp.float32)
        + bl_ref[...]
        + jnp.dot(x_ref[...], wr_ref[...], preferred_element_type=jnp.float32)
    )


def _final(aggp, cntp, x, W_l, b_l, W_r):
    BN = 2000
    return pl.pallas_call(
        _out_body,
        grid=(N // BN,),
        in_specs=[
            pl.BlockSpec((NC, BN, D), lambda i: (0, i, 0)),
            pl.BlockSpec((NC, BN, 16), lambda i: (0, i, 0)),
            pl.BlockSpec((BN, D), lambda i: (i, 0)),
            pl.BlockSpec((D, D), lambda i: (0, 0)),
            pl.BlockSpec((1, D), lambda i: (0, 0)),
            pl.BlockSpec((D, D), lambda i: (0, 0)),
        ],
        out_specs=pl.BlockSpec((BN, D), lambda i: (i, 0)),
        out_shape=jax.ShapeDtypeStruct((N, D), jnp.float32),
    )(aggp, cntp, x, W_l, b_l.reshape(1, D), W_r)


@jax.jit
def kernel(x, edge_attr, edge_index, W_l, b_l, W_r, W_e, b_e):
    ea = _compute_ea(edge_attr, W_e, b_e)
    src_r = edge_index[0].reshape(NW, NB, B)
    dst_r = edge_index[1].reshape(NW, NB, B)
    aggp, cntp = _segment_mean_sums(x, ea, src_r, dst_r)
    out = _final(aggp, cntp, x, W_l, b_l, W_r)
    return (out, ea)


# speedup vs baseline: 1.0203x; 1.0172x over previous
import functools

import jax
import jax.numpy as jnp
from jax import lax
from jax.experimental import pallas as pl
from jax.experimental.pallas import tpu as pltpu
from jax.experimental.pallas import tpu_sc as plsc

N = 10000
E = 320000
D = 128
DE = 16

NC = 2
NS = 16
NW = NC * NS
EW = E // NW
B = 40
NB = EW // B
SB = 125
NSB = NB // SB
ST = 624
TAIL = N - NS * ST


def _ea_body(ea_ref, we_ref, be_ref, out_ref):
    out_ref[...] = (
        jnp.dot(ea_ref[...], we_ref[...], preferred_element_type=jnp.float32)
        + be_ref[...]
    )


def _compute_ea(edge_attr, W_e, b_e):
    BE = 8000
    return pl.pallas_call(
        _ea_body,
        grid=(E // BE,),
        in_specs=[
            pl.BlockSpec((BE, DE), lambda i: (i, 0)),
            pl.BlockSpec((DE, D), lambda i: (0, 0)),
            pl.BlockSpec((1, D), lambda i: (0, 0)),
        ],
        out_specs=pl.BlockSpec((BE, D), lambda i: (i, 0)),
        out_shape=jax.ShapeDtypeStruct((E, D), jnp.float32),
    )(edge_attr, W_e, b_e.reshape(1, D))


def _sc_body(x_hbm, ea_hbm, src_hbm, dst_hbm,
             aggp_hbm, cntp_hbm,
             src_v, dst_v, xj0, xj1, ea0, ea1, ones_v, zb16,
             agg_sh, cnt_sh,
             in_s0, in_s1, out_s0, out_s1):
    c = lax.axis_index("c")
    s = lax.axis_index("s")
    wid = c * NS + s

    xj = (xj0, xj1)
    eab = (ea0, ea1)
    in_s = (in_s0, in_s1)
    out_s = (out_s0, out_s1)

    for i in range(B):
        for j in range(D // 16):
            xj0[i, pl.ds(j * 16, 16)] = jnp.zeros((16,), jnp.float32)
        zb16[i, :] = jnp.zeros((16,), jnp.float32)
        ones_v[i, :] = jnp.ones((16,), jnp.float32)

    NZ = ST // B
    REM = ST - NZ * B
    for k in range(NZ):
        pltpu.sync_copy(xj0, agg_sh.at[pl.ds(s * ST + k * B, B)])
        pltpu.sync_copy(zb16, cnt_sh.at[pl.ds(s * ST + k * B, B)])
    if REM:
        pltpu.sync_copy(xj0.at[pl.ds(0, REM)],
                        agg_sh.at[pl.ds(s * ST + NZ * B, REM)])
        pltpu.sync_copy(zb16.at[pl.ds(0, REM)],
                        cnt_sh.at[pl.ds(s * ST + NZ * B, REM)])

    @pl.when(s == NS - 1)
    def _zero_tail():
        pltpu.sync_copy(xj0.at[pl.ds(0, TAIL)],
                        agg_sh.at[pl.ds(NS * ST, TAIL)])
        pltpu.sync_copy(zb16.at[pl.ds(0, TAIL)],
                        cnt_sh.at[pl.ds(NS * ST, TAIL)])

    plsc.subcore_barrier()

    def start_in(g, p):
        pltpu.async_copy(x_hbm.at[src_v.at[g % SB]], xj[p], in_s[p])
        off = pl.multiple_of(wid * EW + g * B, 8)
        pltpu.async_copy(ea_hbm.at[pl.ds(off, B)], eab[p], in_s[p])

    def wait_in(p):
        pltpu.make_async_copy(x_hbm.at[pl.ds(0, B)], xj[p], in_s[p]).wait()
        pltpu.make_async_copy(x_hbm.at[pl.ds(0, B)], eab[p], in_s[p]).wait()

    def start_out(g, p):
        pltpu.async_copy(xj[p], agg_sh.at[dst_v.at[g % SB]], out_s[p],
                         add=True)
        pltpu.async_copy(ones_v, cnt_sh.at[dst_v.at[g % SB]], out_s[p],
                         add=True)

    def wait_out(p):
        pltpu.make_async_copy(x_hbm.at[pl.ds(0, B)], xj[p], out_s[p]).wait()
        pltpu.make_async_copy(cntp_hbm.at[0, pl.ds(0, B)], ones_v,
                              out_s[p]).wait()

    pltpu.sync_copy(src_hbm.at[wid, pl.ds(0, SB)], src_v)
    pltpu.sync_copy(dst_hbm.at[wid, pl.ds(0, SB)], dst_v)
    start_in(0, 0)

    def phase(g, p):
        q = 1 - p
        wait_in(p)

        @pl.when(g + 1 < NB)
        def _next_in():
            @pl.when((g + 1) % SB == 0)
            def _stage_src():
                pltpu.sync_copy(src_hbm.at[wid, pl.ds(g + 1, SB)], src_v)

            @pl.when(g >= 1)
            def _drain_prev():
                wait_out(q)

            start_in(g + 1, q)

        for i in range(B):
            for j in range(D // 16):
                sl = pl.ds(j * 16, 16)
                xj[p][i, sl] = xj[p][i, sl] * eab[p][i, sl]

        @pl.when(jnp.logical_and(g % SB == 0, g >= 1))
        def _stage_dst():
            pltpu.sync_copy(dst_hbm.at[wid, pl.ds(g, SB)], dst_v)

        start_out(g, p)

    def pair(g2, carry):
        g = g2 * 2
        phase(g, 0)
        phase(g + 1, 1)
        return carry

    lax.fori_loop(0, NB // 2, pair, 0)
    wait_out(0)
    wait_out(1)

    plsc.subcore_barrier()

    pltpu.sync_copy(agg_sh.at[pl.ds(s * ST, ST)],
                    aggp_hbm.at[c, pl.ds(s * ST, ST)])
    pltpu.sync_copy(cnt_sh.at[pl.ds(s * ST, ST)],
                    cntp_hbm.at[c, pl.ds(s * ST, ST)])

    @pl.when(s == NS - 1)
    def _dump_tail():
        pltpu.sync_copy(agg_sh.at[pl.ds(NS * ST, TAIL)],
                        aggp_hbm.at[c, pl.ds(NS * ST, TAIL)])
        pltpu.sync_copy(cnt_sh.at[pl.ds(NS * ST, TAIL)],
                        cntp_hbm.at[c, pl.ds(NS * ST, TAIL)])


def _segment_mean_sums(x, ea, src_r, dst_r):
    mesh = plsc.VectorSubcoreMesh(core_axis_name="c", subcore_axis_name="s")
    f = pl.kernel(
        _sc_body,
        out_type=[
            jax.ShapeDtypeStruct((NC, N, D), jnp.float32),
            jax.ShapeDtypeStruct((NC, N, 16), jnp.float32),
        ],
        mesh=mesh,
        scratch_types=[
            pltpu.VMEM((SB, B), jnp.int32),
            pltpu.VMEM((SB, B), jnp.int32),
            pltpu.VMEM((B, D), jnp.float32),
            pltpu.VMEM((B, D), jnp.float32),
            pltpu.VMEM((B, D), jnp.float32),
            pltpu.VMEM((B, D), jnp.float32),
            pltpu.VMEM((B, 16), jnp.float32),
            pltpu.VMEM((B, 16), jnp.float32),
            pltpu.VMEM_SHARED((N, D), jnp.float32),
            pltpu.VMEM_SHARED((N, 16), jnp.float32),
            pltpu.SemaphoreType.DMA,
            pltpu.SemaphoreType.DMA,
            pltpu.SemaphoreType.DMA,
            pltpu.SemaphoreType.DMA,
        ],
        compiler_params=pltpu.CompilerParams(use_tc_tiling_on_sc=False),
    )
    return f(x, ea, src_r, dst_r)


def _out_body(aggp_ref, cntp_ref, x_ref, wl_ref, bl_ref, wr_ref, out_ref):
    agg = aggp_ref[0] + aggp_ref[1]
    cnt = (cntp_ref[0] + cntp_ref[1]).sum(axis=1) * (1.0 / 16.0)
    agg = agg / jnp.clip(cnt, 1.0)[:, None]
    out_ref[...] = (
        jnp.dot(agg, wl_ref[...], preferred_element_type=jnp.float32)
        + bl_ref[...]
        + jnp.dot(x_ref[...], wr_ref[...], preferred_element_type=jnp.float32)
    )


def _final(aggp, cntp, x, W_l, b_l, W_r):
    BN = 2000
    return pl.pallas_call(
        _out_body,
        grid=(N // BN,),
        in_specs=[
            pl.BlockSpec((NC, BN, D), lambda i: (0, i, 0)),
            pl.BlockSpec((NC, BN, 16), lambda i: (0, i, 0)),
            pl.BlockSpec((BN, D), lambda i: (i, 0)),
            pl.BlockSpec((D, D), lambda i: (0, 0)),
            pl.BlockSpec((1, D), lambda i: (0, 0)),
            pl.BlockSpec((D, D), lambda i: (0, 0)),
        ],
        out_specs=pl.BlockSpec((BN, D), lambda i: (i, 0)),
        out_shape=jax.ShapeDtypeStruct((N, D), jnp.float32),
    )(aggp, cntp, x, W_l, b_l.reshape(1, D), W_r)


@jax.jit
def kernel(x, edge_attr, edge_index, W_l, b_l, W_r, W_e, b_e):
    ea = _compute_ea(edge_attr, W_e, b_e)
    src_r = edge_index[0].reshape(NW, NB, B)
    dst_r = edge_index[1].reshape(NW, NB, B)
    aggp, cntp = _segment_mean_sums(x, ea, src_r, dst_r)
    out = _final(aggp, cntp, x, W_l, b_l, W_r)
    return (out, ea)


# speedup vs baseline: 1.0221x; 1.0017x over previous
import functools

import jax
import jax.numpy as jnp
from jax import lax
from jax.experimental import pallas as pl
from jax.experimental.pallas import tpu as pltpu
from jax.experimental.pallas import tpu_sc as plsc

N = 10000
E = 320000
D = 128
DE = 16

NC = 2
NS = 16
NW = NC * NS
EW = E // NW
B = 40
NB = EW // B
SB = 125
NSB = NB // SB
ST = 624
TAIL = N - NS * ST


def _ea_body(ea_ref, we_ref, be_ref, out_ref):
    out_ref[...] = (
        jnp.dot(ea_ref[...], we_ref[...], preferred_element_type=jnp.float32)
        + be_ref[...]
    )


def _compute_ea(edge_attr, W_e, b_e):
    BE = 16000
    return pl.pallas_call(
        _ea_body,
        grid=(E // BE,),
        in_specs=[
            pl.BlockSpec((BE, DE), lambda i: (i, 0)),
            pl.BlockSpec((DE, D), lambda i: (0, 0)),
            pl.BlockSpec((1, D), lambda i: (0, 0)),
        ],
        out_specs=pl.BlockSpec((BE, D), lambda i: (i, 0)),
        out_shape=jax.ShapeDtypeStruct((E, D), jnp.float32),
    )(edge_attr, W_e, b_e.reshape(1, D))


def _sc_body(x_hbm, ea_hbm, src_hbm, dst_hbm,
             aggp_hbm, cntp_hbm,
             src_v, dst_v, xj0, xj1, ea0, ea1, ones_v, zb16,
             agg_sh, cnt_sh,
             in_s0, in_s1, out_s0, out_s1):
    c = lax.axis_index("c")
    s = lax.axis_index("s")
    wid = c * NS + s

    xj = (xj0, xj1)
    eab = (ea0, ea1)
    in_s = (in_s0, in_s1)
    out_s = (out_s0, out_s1)

    for i in range(B):
        for j in range(D // 16):
            xj0[i, pl.ds(j * 16, 16)] = jnp.zeros((16,), jnp.float32)
        zb16[i, :] = jnp.zeros((16,), jnp.float32)
        ones_v[i, :] = jnp.ones((16,), jnp.float32)

    NZ = ST // B
    REM = ST - NZ * B
    for k in range(NZ):
        pltpu.sync_copy(xj0, agg_sh.at[pl.ds(s * ST + k * B, B)])
        pltpu.sync_copy(zb16, cnt_sh.at[pl.ds(s * ST + k * B, B)])
    if REM:
        pltpu.sync_copy(xj0.at[pl.ds(0, REM)],
                        agg_sh.at[pl.ds(s * ST + NZ * B, REM)])
        pltpu.sync_copy(zb16.at[pl.ds(0, REM)],
                        cnt_sh.at[pl.ds(s * ST + NZ * B, REM)])

    @pl.when(s == NS - 1)
    def _zero_tail():
        pltpu.sync_copy(xj0.at[pl.ds(0, TAIL)],
                        agg_sh.at[pl.ds(NS * ST, TAIL)])
        pltpu.sync_copy(zb16.at[pl.ds(0, TAIL)],
                        cnt_sh.at[pl.ds(NS * ST, TAIL)])

    plsc.subcore_barrier()

    def start_in(g, p):
        pltpu.async_copy(x_hbm.at[src_v.at[g % SB]], xj[p], in_s[p])
        off = pl.multiple_of(wid * EW + g * B, 8)
        pltpu.async_copy(ea_hbm.at[pl.ds(off, B)], eab[p], in_s[p])

    def wait_in(p):
        pltpu.make_async_copy(x_hbm.at[pl.ds(0, B)], xj[p], in_s[p]).wait()
        pltpu.make_async_copy(x_hbm.at[pl.ds(0, B)], eab[p], in_s[p]).wait()

    def start_out(g, p):
        pltpu.async_copy(xj[p], agg_sh.at[dst_v.at[g % SB]], out_s[p],
                         add=True)
        pltpu.async_copy(ones_v, cnt_sh.at[dst_v.at[g % SB]], out_s[p],
                         add=True)

    def wait_out(p):
        pltpu.make_async_copy(x_hbm.at[pl.ds(0, B)], xj[p], out_s[p]).wait()
        pltpu.make_async_copy(cntp_hbm.at[0, pl.ds(0, B)], ones_v,
                              out_s[p]).wait()

    pltpu.sync_copy(src_hbm.at[wid, pl.ds(0, SB)], src_v)
    pltpu.sync_copy(dst_hbm.at[wid, pl.ds(0, SB)], dst_v)
    start_in(0, 0)

    def phase(g, p):
        q = 1 - p
        wait_in(p)

        @pl.when(g + 1 < NB)
        def _next_in():
            @pl.when((g + 1) % SB == 0)
            def _stage_src():
                pltpu.sync_copy(src_hbm.at[wid, pl.ds(g + 1, SB)], src_v)

            @pl.when(g >= 1)
            def _drain_prev():
                wait_out(q)

            start_in(g + 1, q)

        for i in range(B):
            for j in range(D // 16):
                sl = pl.ds(j * 16, 16)
                xj[p][i, sl] = xj[p][i, sl] * eab[p][i, sl]

        @pl.when(jnp.logical_and(g % SB == 0, g >= 1))
        def _stage_dst():
            pltpu.sync_copy(dst_hbm.at[wid, pl.ds(g, SB)], dst_v)

        start_out(g, p)

    def pair(g2, carry):
        g = g2 * 2
        phase(g, 0)
        phase(g + 1, 1)
        return carry

    lax.fori_loop(0, NB // 2, pair, 0)
    wait_out(0)
    wait_out(1)

    plsc.subcore_barrier()

    pltpu.sync_copy(agg_sh.at[pl.ds(s * ST, ST)],
                    aggp_hbm.at[c, pl.ds(s * ST, ST)])
    pltpu.sync_copy(cnt_sh.at[pl.ds(s * ST, ST)],
                    cntp_hbm.at[c, pl.ds(s * ST, ST)])

    @pl.when(s == NS - 1)
    def _dump_tail():
        pltpu.sync_copy(agg_sh.at[pl.ds(NS * ST, TAIL)],
                        aggp_hbm.at[c, pl.ds(NS * ST, TAIL)])
        pltpu.sync_copy(cnt_sh.at[pl.ds(NS * ST, TAIL)],
                        cntp_hbm.at[c, pl.ds(NS * ST, TAIL)])


def _segment_mean_sums(x, ea, src_r, dst_r):
    mesh = plsc.VectorSubcoreMesh(core_axis_name="c", subcore_axis_name="s")
    f = pl.kernel(
        _sc_body,
        out_type=[
            jax.ShapeDtypeStruct((NC, N, D), jnp.float32),
            jax.ShapeDtypeStruct((NC, N, 16), jnp.float32),
        ],
        mesh=mesh,
        scratch_types=[
            pltpu.VMEM((SB, B), jnp.int32),
            pltpu.VMEM((SB, B), jnp.int32),
            pltpu.VMEM((B, D), jnp.float32),
            pltpu.VMEM((B, D), jnp.float32),
            pltpu.VMEM((B, D), jnp.float32),
            pltpu.VMEM((B, D), jnp.float32),
            pltpu.VMEM((B, 16), jnp.float32),
            pltpu.VMEM((B, 16), jnp.float32),
            pltpu.VMEM_SHARED((N, D), jnp.float32),
            pltpu.VMEM_SHARED((N, 16), jnp.float32),
            pltpu.SemaphoreType.DMA,
            pltpu.SemaphoreType.DMA,
            pltpu.SemaphoreType.DMA,
            pltpu.SemaphoreType.DMA,
        ],
        compiler_params=pltpu.CompilerParams(use_tc_tiling_on_sc=False),
    )
    return f(x, ea, src_r, dst_r)


def _out_body(aggp_ref, cntp_ref, x_ref, wl_ref, bl_ref, wr_ref, out_ref):
    agg = aggp_ref[0] + aggp_ref[1]
    cnt = (cntp_ref[0] + cntp_ref[1]).sum(axis=1) * (1.0 / 16.0)
    agg = agg / jnp.clip(cnt, 1.0)[:, None]
    out_ref[...] = (
        jnp.dot(agg, wl_ref[...], preferred_element_type=jnp.float32)
        + bl_ref[...]
        + jnp.dot(x_ref[...], wr_ref[...], preferred_element_type=jnp.float32)
    )


def _final(aggp, cntp, x, W_l, b_l, W_r):
    BN = 2000
    return pl.pallas_call(
        _out_body,
        grid=(N // BN,),
        in_specs=[
            pl.BlockSpec((NC, BN, D), lambda i: (0, i, 0)),
            pl.BlockSpec((NC, BN, 16), lambda i: (0, i, 0)),
            pl.BlockSpec((BN, D), lambda i: (i, 0)),
            pl.BlockSpec((D, D), lambda i: (0, 0)),
            pl.BlockSpec((1, D), lambda i: (0, 0)),
            pl.BlockSpec((D, D), lambda i: (0, 0)),
        ],
        out_specs=pl.BlockSpec((BN, D), lambda i: (i, 0)),
        out_shape=jax.ShapeDtypeStruct((N, D), jnp.float32),
    )(aggp, cntp, x, W_l, b_l.reshape(1, D), W_r)


@jax.jit
def kernel(x, edge_attr, edge_index, W_l, b_l, W_r, W_e, b_e):
    ea = _compute_ea(edge_attr, W_e, b_e)
    src_r = edge_index[0].reshape(NW, NB, B)
    dst_r = edge_index[1].reshape(NW, NB, B)
    aggp, cntp = _segment_mean_sums(x, ea, src_r, dst_r)
    out = _final(aggp, cntp, x, W_l, b_l, W_r)
    return (out, ea)
